# baseline (speedup 1.0000x reference)
import jax
import jax.numpy as jnp
from jax.experimental import pallas as pl
from jax.experimental.pallas import tpu as pltpu

_EPSILON = 1e-4
_N_PAIRS = 4
_NB = 4


def _sinkhorn_compute(x_ref, o_ref):
    s0 = x_ref[...]
    m = jnp.sum(s0, axis=1, keepdims=True)
    v = 1.0 / m
    r = jnp.sum(s0 * v, axis=2, keepdims=True)
    u = 1.0 / (r + _EPSILON)
    for _ in range(_N_PAIRS):
        m = jnp.sum(s0 * u, axis=1, keepdims=True)
        v = 1.0 / m
        r = jnp.sum(s0 * v, axis=2, keepdims=True)
        u = u / (u * r + _EPSILON)
    o_ref[...] = s0 * u * v


def _pipeline_body(n_steps, s_hbm, o_hbm, x_buf, o_buf, in_sem, out_sem):
    base = pl.program_id(0) * (n_steps * _NB)

    def dma_in(slot, step):
        pltpu.make_async_copy(
            s_hbm.at[pl.ds(base + step * _NB, _NB)], x_buf.at[slot],
            in_sem.at[slot]).start()

    def wait_in(slot):
        pltpu.make_async_copy(
            s_hbm.at[pl.ds(0, _NB)], x_buf.at[slot],
            in_sem.at[slot]).wait()

    def dma_out(slot, step):
        pltpu.make_async_copy(
            o_buf.at[slot], o_hbm.at[pl.ds(base + step * _NB, _NB)],
            out_sem.at[slot]).start(priority=1)

    def wait_out(slot):
        pltpu.make_async_copy(
            o_buf.at[slot], o_hbm.at[pl.ds(0, _NB)],
            out_sem.at[slot]).wait()

    dma_in(0, 0)

    def body(step, _):
        cur = jax.lax.rem(step, 2)
        nxt = jax.lax.rem(step + 1, 2)

        @pl.when(step + 1 < n_steps)
        def _():
            dma_in(nxt, step + 1)

        wait_in(cur)

        @pl.when(step >= 2)
        def _():
            wait_out(cur)

        _sinkhorn_compute(x_buf.at[cur], o_buf.at[cur])
        dma_out(cur, step)
        return ()

    jax.lax.fori_loop(0, n_steps, body, ())
    wait_out(jax.lax.rem(n_steps - 2, 2))
    wait_out(jax.lax.rem(n_steps - 1, 2))


def kernel(s):
    b, n1, n2 = s.shape
    n_cores = 2
    n_steps = b // (_NB * n_cores)

    def body(s_hbm, o_hbm, x_buf, o_buf, in_sem, out_sem):
        _pipeline_body(n_steps, s_hbm, o_hbm, x_buf, o_buf, in_sem, out_sem)

    return pl.pallas_call(
        body,
        grid=(n_cores,),
        in_specs=[pl.BlockSpec(memory_space=pl.ANY)],
        out_specs=pl.BlockSpec(memory_space=pl.ANY),
        out_shape=jax.ShapeDtypeStruct(s.shape, s.dtype),
        scratch_shapes=[
            pltpu.VMEM((2, _NB, n1, n2), s.dtype),
            pltpu.VMEM((2, _NB, n1, n2), s.dtype),
            pltpu.SemaphoreType.DMA((2,)),
            pltpu.SemaphoreType.DMA((2,)),
        ],
        compiler_params=pltpu.CompilerParams(
            dimension_semantics=("parallel",),
        ),
    )(s)


# speedup vs baseline: 1.0595x; 1.0595x over previous
import jax
import jax.numpy as jnp
from jax.experimental import pallas as pl
from jax.experimental.pallas import tpu as pltpu

_EPSILON = 1e-4
_N_PAIRS = 4


def _sinkhorn_body(s_ref, o_ref):
    s0 = s_ref[...]
    m = jnp.sum(s0, axis=1, keepdims=True)
    v = 1.0 / m
    r = jnp.sum(s0 * v, axis=2, keepdims=True)
    u = 1.0 / (r + _EPSILON)
    for _ in range(_N_PAIRS):
        m = jax.lax.dot_general(u, s0, (((1,), (1,)), ((0,), (0,))),
                                preferred_element_type=jnp.float32)
        v = 1.0 / m
        r = jnp.sum(s0 * v, axis=2, keepdims=True)
        u = u / (u * r + _EPSILON)
    o_ref[...] = s0 * u * v


def kernel(s):
    b, n1, n2 = s.shape
    return pl.pallas_call(
        _sinkhorn_body,
        grid=(b // 4,),
        in_specs=[pl.BlockSpec((4, n1, n2), lambda i: (i, 0, 0))],
        out_specs=pl.BlockSpec((4, n1, n2), lambda i: (i, 0, 0)),
        out_shape=jax.ShapeDtypeStruct(s.shape, s.dtype),
        compiler_params=pltpu.CompilerParams(
            dimension_semantics=("parallel",),
        ),
    )(s)


# speedup vs baseline: 1.0678x; 1.0078x over previous
import jax
import jax.numpy as jnp
from jax.experimental import pallas as pl
from jax.experimental.pallas import tpu as pltpu

_EPSILON = 1e-4
_N_PAIRS = 4


def _sinkhorn_body(s_ref, o_ref):
    s0 = s_ref[...]
    m = jnp.sum(s0, axis=1, keepdims=True)
    v = 1.0 / m
    r = jnp.sum(s0 * v, axis=2, keepdims=True)
    w = r + _EPSILON
    u = 1.0 / w
    for _ in range(_N_PAIRS):
        m = jax.lax.dot_general(u, s0, (((1,), (1,)), ((0,), (0,))),
                                preferred_element_type=jnp.float32)
        v = 1.0 / m
        r = jnp.sum(s0 * v, axis=2, keepdims=True)
        w = r + _EPSILON * w
        u = 1.0 / w
    o_ref[...] = s0 * u * v


def kernel(s):
    b, n1, n2 = s.shape
    return pl.pallas_call(
        _sinkhorn_body,
        grid=(b // 4,),
        in_specs=[pl.BlockSpec((4, n1, n2), lambda i: (i, 0, 0))],
        out_specs=pl.BlockSpec((4, n1, n2), lambda i: (i, 0, 0)),
        out_shape=jax.ShapeDtypeStruct(s.shape, s.dtype),
        compiler_params=pltpu.CompilerParams(
            dimension_semantics=("parallel",),
        ),
    )(s)


# speedup vs baseline: 1.1932x; 1.1174x over previous
import jax
import jax.numpy as jnp
from jax.experimental import pallas as pl
from jax.experimental.pallas import tpu as pltpu

_EPSILON = 1e-4
_N_PAIRS = 4


def _sinkhorn_body(s_ref, o_ref):
    s0 = s_ref[...]
    m = jnp.sum(s0, axis=1, keepdims=True)
    v = 1.0 / m
    r = jnp.sum(s0 * v, axis=2, keepdims=True)
    w = r + _EPSILON
    u = 1.0 / w
    for _ in range(_N_PAIRS):
        m = jax.lax.dot_general(u, s0, (((1,), (1,)), ((0,), (0,))),
                                preferred_element_type=jnp.float32)
        v = 1.0 / m
        r = jnp.sum(s0 * v, axis=2, keepdims=True)
        w = r + _EPSILON * w
        u = 1.0 / w
    o_ref[...] = s0 * u * v


def kernel(s):
    b, n1, n2 = s.shape
    return pl.pallas_call(
        _sinkhorn_body,
        grid=(b // 8,),
        in_specs=[pl.BlockSpec((8, n1, n2), lambda i: (i, 0, 0))],
        out_specs=pl.BlockSpec((8, n1, n2), lambda i: (i, 0, 0)),
        out_shape=jax.ShapeDtypeStruct(s.shape, s.dtype),
        compiler_params=pltpu.CompilerParams(
            dimension_semantics=("parallel",),
        ),
    )(s)


# speedup vs baseline: 1.1974x; 1.0035x over previous
import jax
import jax.numpy as jnp
from jax.experimental import pallas as pl
from jax.experimental.pallas import tpu as pltpu

_EPSILON = 1e-4
_N_PAIRS = 4


def _sinkhorn_body(s_ref, o_ref):
    s0 = s_ref[...]
    m = jnp.sum(s0, axis=1, keepdims=True)
    v = 1.0 / m
    r = jnp.sum(s0 * v, axis=2, keepdims=True)
    w = r + _EPSILON
    u = 1.0 / w
    for _ in range(_N_PAIRS - 1):
        m = jax.lax.dot_general(u, s0, (((1,), (1,)), ((0,), (0,))),
                                preferred_element_type=jnp.float32)
        v = 1.0 / m
        r = jnp.sum(s0 * v, axis=2, keepdims=True)
        w = r + _EPSILON * w
        u = 1.0 / w
    m = jax.lax.dot_general(u, s0, (((1,), (1,)), ((0,), (0,))),
                            preferred_element_type=jnp.float32)
    v = 1.0 / m
    t = s0 * v
    r = jnp.sum(t, axis=2, keepdims=True)
    w = r + _EPSILON * w
    u = 1.0 / w
    o_ref[...] = t * u


def kernel(s):
    b, n1, n2 = s.shape
    return pl.pallas_call(
        _sinkhorn_body,
        grid=(b // 8,),
        in_specs=[pl.BlockSpec((8, n1, n2), lambda i: (i, 0, 0))],
        out_specs=pl.BlockSpec((8, n1, n2), lambda i: (i, 0, 0)),
        out_shape=jax.ShapeDtypeStruct(s.shape, s.dtype),
        compiler_params=pltpu.CompilerParams(
            dimension_semantics=("parallel",),
        ),
    )(s)
